# Initial kernel scaffold; baseline (speedup 1.0000x reference)
import jax
import jax.numpy as jnp
from jax.experimental import pallas as pl


def kernel(tokens, spatial_xyz, W1, b1, W2, b2, centers, t):
    raise NotImplementedError("write your pallas kernel here")


# speedup vs baseline: 2.5477x; 2.5477x over previous
import functools

import jax
import jax.numpy as jnp
from jax import lax
from jax.experimental import pallas as pl
from jax.experimental.pallas import tpu as pltpu

B, N, D, E = 2, 2048, 1024, 8
TOP_K = 2
NUM_EXPERTS_B = 4
FLOOR = min(0.05, 0.15 / max(NUM_EXPERTS_B, 1))
ALPHA = min(FLOOR * E, 1.0)
CAP_LOW, CAP_HIGH, T_MAX = 0.5, 0.6, 1000
NT = B * N
H = D // 2
K_SEL = min(max(1, N * TOP_K // E), N)
BT = 512
NBLK = NT // BT
INT_MIN = -2147483648


def _router_body(tok_ref, xyzT_ref, xyzTb_ref, W1at_ref, W1bt_ref, b1_ref,
                 W2t_ref, b2_ref, centers_ref, t_ref, dT_ref, cT_ref,
                 distsT_s, logitsT_s, eye_s, inv_s):
    i = pl.program_id(0)

    @pl.when(i == 0)
    def _prologue():
        r = lax.broadcasted_iota(jnp.int32, (BT, BT), 0)
        c = lax.broadcasted_iota(jnp.int32, (BT, BT), 1)
        eye_s[...] = (r == c).astype(jnp.float32)
        d2 = jnp.zeros((E, NT), jnp.float32)
        for j in range(3):
            xr = xyzT_ref[j:j + 1, :]
            cc = centers_ref[:, j:j + 1]
            diff = cc - xr
            d2 = d2 + diff * diff
        dTmat = jnp.sqrt(d2)
        distsT_s[...] = dTmat
        mean = jnp.sum(dTmat) / (B * N * E)
        inv_s[0, 0] = 1.0 / (mean + 1e-6)

    inv = inv_s[0, 0]
    xslice = xyzTb_ref[:, pl.ds(i * BT, BT)]
    pre = jnp.dot(tok_ref[...], W1at_ref[...],
                  preferred_element_type=jnp.float32)
    pre = pre + lax.dot_general(xslice, W1bt_ref[...],
                                (((0,), (0,)), ((), ())),
                                preferred_element_type=jnp.float32)
    pre = pre + b1_ref[...]
    h = 0.5 * pre * (1.0 + lax.erf(pre * (2.0 ** -0.5)))
    content = jnp.dot(h.astype(jnp.bfloat16), W2t_ref[...],
                      preferred_element_type=jnp.float32) + b2_ref[...]
    contentT = lax.dot_general(content, eye_s[...],
                               (((0,), (0,)), ((), ())),
                               precision=lax.Precision.HIGHEST,
                               preferred_element_type=jnp.float32)
    aff = distsT_s[:, pl.ds(i * BT, BT)] * (-inv)
    lT = contentT + aff
    b = i // (N // BT)
    off = (i % (N // BT)) * BT
    logitsT_s[pl.ds(b * E, E), pl.ds(off, BT)] = lT

    @pl.when(i == NBLK - 1)
    def _epilogue():
        L = logitsT_s[...]
        bits = lax.bitcast_convert_type(L, jnp.int32)
        key = jnp.where(bits < 0, jnp.int32(INT_MIN) - bits, bits)
        kk = jnp.float32(K_SEL)

        def cnt_ge(cand):
            return jnp.sum((key >= cand).astype(jnp.float32),
                           axis=1, keepdims=True)

        T0 = jnp.where(cnt_ge(jnp.zeros((B * E, 1), jnp.int32)) >= kk,
                       jnp.int32(0), jnp.int32(INT_MIN))

        def bs_body(it, T):
            bit = 30 - it
            cand = T | lax.shift_left(jnp.int32(1), bit)
            return jnp.where(cnt_ge(cand) >= kk, cand, T)

        T = lax.fori_loop(0, 31, bs_body, T0)
        c_gt = jnp.sum((key > T).astype(jnp.float32), axis=1, keepdims=True)
        r = kk - c_gt
        idx = lax.broadcasted_iota(jnp.int32, (B * E, N), 1)
        eqT = key == T

        def jp_body(it, J):
            bit = 11 - it
            cand = J | lax.shift_left(jnp.int32(1), bit)
            hp = jnp.sum((eqT & (idx < cand)).astype(jnp.float32),
                         axis=1, keepdims=True)
            ok = (cand <= N) & (hp < r)
            return jnp.where(ok, cand, J)

        Jp = lax.fori_loop(0, 12, jp_body, jnp.zeros((B * E, 1), jnp.int32))
        sel = (key > T) | (eqT & (idx <= Jp))
        d0 = jnp.where(sel, jax.nn.sigmoid(L), 0.0)
        d1 = (1.0 - ALPHA) * d0 + (ALPHA / E)

        t0 = t_ref[0].astype(jnp.float32)
        t1 = t_ref[1].astype(jnp.float32)
        cap0 = CAP_LOW + (CAP_HIGH + CAP_LOW) * (t0 / T_MAX)
        cap1 = CAP_LOW + (CAP_HIGH + CAP_LOW) * (t1 / T_MAX)
        riota = lax.broadcasted_iota(jnp.int32, (B * E, 1), 0)
        cap = jnp.where(riota < E, cap0, cap1)

        def bsum(X):
            s0 = jnp.sum(X[0:E, :], axis=0, keepdims=True)
            s1 = jnp.sum(X[E:2 * E, :], axis=0, keepdims=True)
            return jnp.concatenate([jnp.broadcast_to(s0, (E, N)),
                                    jnp.broadcast_to(s1, (E, N))], axis=0)

        excess = jnp.maximum(d1 - cap, 0.0)
        capped = d1 - excess
        headroom = jnp.maximum(cap - capped, 0.0)
        hsum = jnp.maximum(bsum(headroom), 1e-8)
        capped = capped + bsum(excess) * (headroom / hsum)
        dT_ref[...] = capped
        cT_ref[...] = capped / (bsum(capped) + 1e-8)


@jax.jit
def kernel(tokens, spatial_xyz, W1, b1, W2, b2, centers, t):
    tok = tokens.reshape(NT, D).astype(jnp.bfloat16)
    xyzT = spatial_xyz.reshape(NT, 3).T
    xyzTb = xyzT.astype(jnp.bfloat16)
    W1at = W1[:, :D].T.astype(jnp.bfloat16)
    W1bt = W1[:, D:].T.astype(jnp.bfloat16)
    W2t = W2.T.astype(jnp.bfloat16)
    b1r = b1.reshape(1, H)
    b2r = b2.reshape(1, E)

    dT, cT = pl.pallas_call(
        _router_body,
        grid=(NBLK,),
        in_specs=[
            pl.BlockSpec((BT, D), lambda i: (i, 0)),
            pl.BlockSpec((3, NT), lambda i: (0, 0)),
            pl.BlockSpec((3, NT), lambda i: (0, 0)),
            pl.BlockSpec((D, H), lambda i: (0, 0)),
            pl.BlockSpec((3, H), lambda i: (0, 0)),
            pl.BlockSpec((1, H), lambda i: (0, 0)),
            pl.BlockSpec((H, E), lambda i: (0, 0)),
            pl.BlockSpec((1, E), lambda i: (0, 0)),
            pl.BlockSpec((E, 3), lambda i: (0, 0)),
            pl.BlockSpec(memory_space=pltpu.SMEM),
        ],
        out_specs=[
            pl.BlockSpec((B * E, N), lambda i: (0, 0)),
            pl.BlockSpec((B * E, N), lambda i: (0, 0)),
        ],
        out_shape=[
            jax.ShapeDtypeStruct((B * E, N), jnp.float32),
            jax.ShapeDtypeStruct((B * E, N), jnp.float32),
        ],
        scratch_shapes=[
            pltpu.VMEM((E, NT), jnp.float32),
            pltpu.VMEM((B * E, N), jnp.float32),
            pltpu.VMEM((BT, BT), jnp.float32),
            pltpu.SMEM((1, 1), jnp.float32),
        ],
    )(tok, xyzT, xyzTb, W1at, W1bt, b1r, W2t, b2r, centers, t)

    dispatch = dT.reshape(B, E, N).transpose(0, 2, 1)
    combine = cT.reshape(B, E, N).transpose(0, 2, 1)
    return (dispatch, combine)
